# Initial kernel scaffold; baseline (speedup 1.0000x reference)
import jax
import jax.numpy as jnp
from jax.experimental import pallas as pl


def kernel(cell_types, table):
    raise NotImplementedError("write your pallas kernel here")


# speedup vs baseline: 5.1070x; 5.1070x over previous
import functools

import jax
import jax.numpy as jnp
from jax import lax
from jax.experimental import pallas as pl
from jax.experimental.pallas import tpu as pltpu
from jax.experimental.pallas import tpu_sc as plsc


def _make_gather(V, D, B):
    info = plsc.get_sparse_core_info()
    NC, NS = info.num_cores, info.num_subcores
    NW = NC * NS
    assert B % NW == 0
    b_per_w = B // NW
    CH = 1280
    assert b_per_w % CH == 0
    n_chunks = b_per_w // CH

    mesh = plsc.VectorSubcoreMesh(core_axis_name="c", subcore_axis_name="s")

    @functools.partial(
        pl.kernel,
        mesh=mesh,
        out_type=jax.ShapeDtypeStruct((B, D), jnp.float32),
        scratch_types=[
            pltpu.VMEM((CH,), jnp.int32),
            pltpu.VMEM((CH, D), jnp.float32),
            pltpu.SemaphoreType.DMA,
        ],
        compiler_params=pltpu.CompilerParams(use_tc_tiling_on_sc=False),
    )
    def gather_kernel(table_hbm, idx_hbm, out_hbm, idx_v, rows_v, sem):
        wid = lax.axis_index("s") * NC + lax.axis_index("c")
        base = wid * b_per_w

        def body(i, carry):
            off = base + i * CH
            pltpu.sync_copy(idx_hbm.at[pl.ds(off, CH)], idx_v)
            pltpu.async_copy(table_hbm.at[idx_v], rows_v, sem).wait()
            pltpu.sync_copy(rows_v, out_hbm.at[pl.ds(off, CH)])
            return carry

        lax.fori_loop(0, n_chunks, body, 0)

    return gather_kernel


def kernel(cell_types, table):
    B0, B1 = cell_types.shape
    V, D = table.shape
    idx = cell_types.reshape(-1).astype(jnp.int32)
    out = _make_gather(V, D, B0 * B1)(table, idx)
    return out.reshape(B0, B1, D)


# speedup vs baseline: 5.3039x; 1.0385x over previous
import functools

import jax
import jax.numpy as jnp
from jax import lax
from jax.experimental import pallas as pl
from jax.experimental.pallas import tpu as pltpu
from jax.experimental.pallas import tpu_sc as plsc


def _make_gather(V, D, B):
    info = plsc.get_sparse_core_info()
    NC, NS = info.num_cores, info.num_subcores
    NW = NC * NS
    assert B % NW == 0
    b_per_w = B // NW
    CH = 1280
    assert b_per_w % (2 * CH) == 0
    n_half = b_per_w // (2 * CH)

    mesh = plsc.VectorSubcoreMesh(core_axis_name="c", subcore_axis_name="s")

    @functools.partial(
        pl.kernel,
        mesh=mesh,
        out_type=jax.ShapeDtypeStruct((B, D), jnp.float32),
        scratch_types=[
            pltpu.VMEM((b_per_w,), jnp.int32),
            pltpu.VMEM((CH, D), jnp.float32),
            pltpu.VMEM((CH, D), jnp.float32),
            pltpu.SemaphoreType.DMA,
            pltpu.SemaphoreType.DMA,
        ],
        compiler_params=pltpu.CompilerParams(use_tc_tiling_on_sc=False),
    )
    def gather_kernel(table_hbm, idx_hbm, out_hbm, idx_v, rows_a, rows_b,
                      sem_a, sem_b):
        wid = lax.axis_index("s") * NC + lax.axis_index("c")
        base = wid * b_per_w
        pltpu.sync_copy(idx_hbm.at[pl.ds(base, b_per_w)], idx_v)

        def start_gather(j, buf, sem):
            pltpu.async_copy(table_hbm.at[idx_v.at[pl.ds(j * CH, CH)]],
                             buf, sem)

        def wait_gather(j, buf, sem):
            pltpu.make_async_copy(table_hbm.at[idx_v.at[pl.ds(j * CH, CH)]],
                                  buf, sem).wait()

        def store(j, buf):
            pltpu.sync_copy(buf, out_hbm.at[pl.ds(base + j * CH, CH)])

        start_gather(0, rows_a, sem_a)

        def body(i, carry):
            ka = 2 * i
            kb = 2 * i + 1
            start_gather(kb, rows_b, sem_b)
            wait_gather(ka, rows_a, sem_a)
            store(ka, rows_a)

            @pl.when(i + 1 < n_half)
            def _():
                start_gather(ka + 2, rows_a, sem_a)

            wait_gather(kb, rows_b, sem_b)
            store(kb, rows_b)
            return carry

        lax.fori_loop(0, n_half, body, 0)

    return gather_kernel


def kernel(cell_types, table):
    B0, B1 = cell_types.shape
    V, D = table.shape
    idx = cell_types.reshape(-1).astype(jnp.int32)
    out = _make_gather(V, D, B0 * B1)(table, idx)
    return out.reshape(B0, B1, D)


# speedup vs baseline: 15.5638x; 2.9344x over previous
import functools

import jax
import jax.numpy as jnp
from jax import lax
from jax.experimental import pallas as pl
from jax.experimental.pallas import tpu as pltpu
from jax.experimental.pallas import tpu_sc as plsc


def _make_gather_t(V, D, B0, B1):
    info = plsc.get_sparse_core_info()
    NC, NS, L = info.num_cores, info.num_subcores, info.num_lanes
    assert NC * NS == D and B0 % (2 * L) == 0 and B1 % 2 == 0
    n_groups = B0 // L

    mesh = plsc.VectorSubcoreMesh(core_axis_name="c", subcore_axis_name="s")

    @functools.partial(
        pl.kernel,
        mesh=mesh,
        out_type=jax.ShapeDtypeStruct((B1, D, B0), jnp.float32),
        scratch_types=[
            pltpu.VMEM((V,), jnp.float32),
            pltpu.VMEM((B0,), jnp.int32),
            pltpu.VMEM((B0,), jnp.int32),
            pltpu.VMEM((B0,), jnp.float32),
            pltpu.VMEM((B0,), jnp.float32),
            pltpu.SemaphoreType.DMA,
            pltpu.SemaphoreType.DMA,
            pltpu.SemaphoreType.DMA,
            pltpu.SemaphoreType.DMA,
        ],
        compiler_params=pltpu.CompilerParams(use_tc_tiling_on_sc=True,
                                             needs_layout_passes=False),
    )
    def gather_kernel(table_hbm, ct_hbm, out_hbm, row_v, idx_a, idx_b,
                      out_a, out_b, sia, sib, soa, sob):
        k = lax.axis_index("c") * NS + lax.axis_index("s")
        pltpu.sync_copy(table_hbm.at[k], row_v)

        def start_idx(j, buf, sem):
            pltpu.async_copy(ct_hbm.at[j], buf, sem)

        def wait_idx(j, buf, sem):
            pltpu.make_async_copy(ct_hbm.at[j], buf, sem).wait()

        def start_out(j, buf, sem):
            pltpu.async_copy(buf, out_hbm.at[j, k], sem)

        def wait_out(j, buf, sem):
            pltpu.make_async_copy(buf, out_hbm.at[j, k], sem).wait()

        def gather_row(idx_v, out_v):
            @plsc.parallel_loop(0, n_groups, unroll=8)
            def _(g):
                idx16 = idx_v[pl.ds(g * L, L)]
                out_v[pl.ds(g * L, L)] = plsc.load_gather(row_v, [idx16])

        start_idx(0, idx_a, sia)
        start_idx(1, idx_b, sib)

        def body(i, carry):
            j0 = 2 * i
            j1 = 2 * i + 1

            wait_idx(j0, idx_a, sia)

            @pl.when(i > 0)
            def _():
                wait_out(j0 - 2, out_a, soa)

            gather_row(idx_a, out_a)
            start_out(j0, out_a, soa)

            @pl.when(j0 + 2 < B1)
            def _():
                start_idx(j0 + 2, idx_a, sia)

            wait_idx(j1, idx_b, sib)

            @pl.when(i > 0)
            def _():
                wait_out(j1 - 2, out_b, sob)

            gather_row(idx_b, out_b)
            start_out(j1, out_b, sob)

            @pl.when(j1 + 2 < B1)
            def _():
                start_idx(j1 + 2, idx_b, sib)

            return carry

        lax.fori_loop(0, B1 // 2, body, 0)
        wait_out(B1 - 2, out_a, soa)
        wait_out(B1 - 1, out_b, sob)

    return gather_kernel


def kernel(cell_types, table):
    B0, B1 = cell_types.shape
    V, D = table.shape
    ct_t = jnp.transpose(cell_types.astype(jnp.int32))
    table_t = jnp.transpose(table)
    out_t = _make_gather_t(V, D, B0, B1)(table_t, ct_t)
    return jnp.transpose(out_t, (2, 0, 1))


# speedup vs baseline: 18.1748x; 1.1678x over previous
import functools

import jax
import jax.numpy as jnp
from jax import lax
from jax.experimental import pallas as pl
from jax.experimental.pallas import tpu as pltpu
from jax.experimental.pallas import tpu_sc as plsc

_NBUF = 3


def _make_gather_t(V, D, B0, B1):
    info = plsc.get_sparse_core_info()
    NC, NS, L = info.num_cores, info.num_subcores, info.num_lanes
    assert NC * NS == D and B0 % L == 0
    n_groups = B0 // L
    n_main = (B1 // (2 * _NBUF)) * (2 * _NBUF)
    tail = list(range(n_main, B1))

    mesh = plsc.VectorSubcoreMesh(core_axis_name="c", subcore_axis_name="s")

    @functools.partial(
        pl.kernel,
        mesh=mesh,
        out_type=jax.ShapeDtypeStruct((B1, D, B0), jnp.float32),
        scratch_types=[
            pltpu.VMEM((V,), jnp.float32),
            [pltpu.VMEM((B0,), jnp.int32) for _ in range(_NBUF)],
            [pltpu.VMEM((B0,), jnp.float32) for _ in range(_NBUF)],
            [pltpu.SemaphoreType.DMA for _ in range(_NBUF)],
            [pltpu.SemaphoreType.DMA for _ in range(_NBUF)],
        ],
        compiler_params=pltpu.CompilerParams(use_tc_tiling_on_sc=True,
                                             needs_layout_passes=False),
    )
    def gather_kernel(table_hbm, ct_hbm, out_hbm, row_v, idx_bufs, out_bufs,
                      idx_sems, out_sems):
        k = lax.axis_index("c") * NS + lax.axis_index("s")
        pltpu.sync_copy(table_hbm.at[k], row_v)

        def start_idx(j, b):
            pltpu.async_copy(ct_hbm.at[j], idx_bufs[b], idx_sems[b])

        def wait_idx(j, b):
            pltpu.make_async_copy(ct_hbm.at[j], idx_bufs[b],
                                  idx_sems[b]).wait()

        def start_out(j, b):
            pltpu.async_copy(out_bufs[b], out_hbm.at[j, k], out_sems[b])

        def wait_out(j, b):
            pltpu.make_async_copy(out_bufs[b], out_hbm.at[j, k],
                                  out_sems[b]).wait()

        def gather_row(b):
            idx_v, out_v = idx_bufs[b], out_bufs[b]

            @plsc.parallel_loop(0, n_groups, unroll=8)
            def _(g):
                idx16 = idx_v[pl.ds(g * L, L)]
                out_v[pl.ds(g * L, L)] = plsc.load_gather(row_v, [idx16])

        for b in range(_NBUF):
            start_idx(b, b)

        def body(i, carry):
            for p in range(2 * _NBUF):
                b = p % _NBUF
                j = 2 * _NBUF * i + p
                wait_idx(j, b)
                if p >= _NBUF:
                    wait_out(j - _NBUF, b)
                else:
                    @pl.when(i > 0)
                    def _():
                        wait_out(j - _NBUF, b)
                gather_row(b)
                start_out(j, b)

                @pl.when(j + _NBUF < B1)
                def _():
                    start_idx(j + _NBUF, b)
            return carry

        lax.fori_loop(0, n_main // (2 * _NBUF), body, 0)

        for j in tail:
            b = j % _NBUF
            wait_idx(j, b)
            wait_out(j - _NBUF, b)
            gather_row(b)
            start_out(j, b)
        for j in range(B1 - _NBUF, B1):
            wait_out(j, j % _NBUF)

    return gather_kernel


def kernel(cell_types, table):
    B0, B1 = cell_types.shape
    V, D = table.shape
    ct_t = jnp.transpose(cell_types.astype(jnp.int32))
    table_t = jnp.transpose(table)
    out_t = _make_gather_t(V, D, B0, B1)(table_t, ct_t)
    return jnp.transpose(out_t, (2, 0, 1))


# speedup vs baseline: 18.1809x; 1.0003x over previous
import functools

import jax
import jax.numpy as jnp
from jax import lax
from jax.experimental import pallas as pl
from jax.experimental.pallas import tpu as pltpu
from jax.experimental.pallas import tpu_sc as plsc

_NBUF = 3


def _make_gather_t(V, D, B0, B1):
    info = plsc.get_sparse_core_info()
    NC, NS, L = info.num_cores, info.num_subcores, info.num_lanes
    assert NC * NS == D and B0 % L == 0
    n_groups = B0 // L
    n_main = (B1 // (2 * _NBUF)) * (2 * _NBUF)
    tail = list(range(n_main, B1))

    mesh = plsc.VectorSubcoreMesh(core_axis_name="c", subcore_axis_name="s")

    @functools.partial(
        pl.kernel,
        mesh=mesh,
        out_type=jax.ShapeDtypeStruct((B1, D, B0), jnp.float32),
        scratch_types=[
            pltpu.VMEM((V,), jnp.float32),
            [pltpu.VMEM((B0,), jnp.int32) for _ in range(_NBUF)],
            [pltpu.VMEM((B0,), jnp.float32) for _ in range(_NBUF)],
            [pltpu.SemaphoreType.DMA for _ in range(_NBUF)],
            [pltpu.SemaphoreType.DMA for _ in range(_NBUF)],
        ],
        compiler_params=pltpu.CompilerParams(use_tc_tiling_on_sc=True,
                                             needs_layout_passes=False),
    )
    def gather_kernel(table_hbm, ct_hbm, out_hbm, row_v, idx_bufs, out_bufs,
                      idx_sems, out_sems):
        k = lax.axis_index("c") * NS + lax.axis_index("s")
        pltpu.sync_copy(table_hbm.at[k], row_v)

        def start_idx(j, b):
            pltpu.async_copy(ct_hbm.at[j], idx_bufs[b], idx_sems[b])

        def wait_idx(j, b):
            pltpu.make_async_copy(ct_hbm.at[j], idx_bufs[b],
                                  idx_sems[b]).wait()

        def start_out(j, b):
            pltpu.async_copy(out_bufs[b], out_hbm.at[j, k], out_sems[b])

        def wait_out(j, b):
            pltpu.make_async_copy(out_bufs[b], out_hbm.at[j, k],
                                  out_sems[b]).wait()

        def gather_row(b):
            idx_v, out_v = idx_bufs[b], out_bufs[b]

            @plsc.parallel_loop(0, n_groups, unroll=16)
            def _(g):
                idx16 = idx_v[pl.ds(g * L, L)]
                out_v[pl.ds(g * L, L)] = plsc.load_gather(row_v, [idx16])

        for b in range(_NBUF):
            start_idx(b, b)

        def body(i, carry):
            for p in range(2 * _NBUF):
                b = p % _NBUF
                j = 2 * _NBUF * i + p
                wait_idx(j, b)
                if p >= _NBUF:
                    wait_out(j - _NBUF, b)
                else:
                    @pl.when(i > 0)
                    def _():
                        wait_out(j - _NBUF, b)
                gather_row(b)
                start_out(j, b)

                @pl.when(j + _NBUF < B1)
                def _():
                    start_idx(j + _NBUF, b)
            return carry

        lax.fori_loop(0, n_main // (2 * _NBUF), body, 0)

        for j in tail:
            b = j % _NBUF
            wait_idx(j, b)
            wait_out(j - _NBUF, b)
            gather_row(b)
            start_out(j, b)
        for j in range(B1 - _NBUF, B1):
            wait_out(j, j % _NBUF)

    return gather_kernel


def kernel(cell_types, table):
    B0, B1 = cell_types.shape
    V, D = table.shape
    ct_t = jnp.transpose(cell_types.astype(jnp.int32))
    table_t = jnp.transpose(table)
    out_t = _make_gather_t(V, D, B0, B1)(table_t, ct_t)
    return jnp.transpose(out_t, (2, 0, 1))


# speedup vs baseline: 27.2561x; 1.4992x over previous
import functools

import jax
import jax.numpy as jnp
from jax import lax
from jax.experimental import pallas as pl
from jax.experimental.pallas import tpu as pltpu
from jax.experimental.pallas import tpu_sc as plsc

_NBUF = 3
_LEAD = 3
_RING = 6


def _make_gather_t(V, D, B0, B1):
    info = plsc.get_sparse_core_info()
    NC, NS, L = info.num_cores, info.num_subcores, info.num_lanes
    assert NC * NS == D and B0 % L == 0
    n_groups = B0 // L

    mesh = plsc.VectorSubcoreMesh(core_axis_name="c", subcore_axis_name="s")

    @functools.partial(
        pl.kernel,
        mesh=mesh,
        out_type=jax.ShapeDtypeStruct((B1, D, B0), jnp.float32),
        scratch_types=[
            pltpu.VMEM((V,), jnp.float32),
            [pltpu.VMEM((B0,), jnp.int32) for _ in range(_NBUF)],
            [pltpu.VMEM((B0,), jnp.float32) for _ in range(_NBUF)],
            pltpu.VMEM_SHARED((_RING * B0,), jnp.int32),
            [pltpu.SemaphoreType.DMA for _ in range(_NBUF)],
            [pltpu.SemaphoreType.DMA for _ in range(_NBUF)],
            pltpu.SemaphoreType.DMA,
        ],
        compiler_params=pltpu.CompilerParams(use_tc_tiling_on_sc=True,
                                             needs_layout_passes=False),
    )
    def gather_kernel(table_hbm, ct_hbm, out_hbm, row_v, idx_bufs, out_bufs,
                      ct_ring, idx_sems, out_sems, stage_sem):
        s = lax.axis_index("s")
        k = lax.axis_index("c") * NS + s

        def ring_slot(j):
            return ct_ring.at[pl.ds((j % _RING) * B0, B0)]

        def start_stage(j):
            pltpu.async_copy(ct_hbm.at[j], ring_slot(j), stage_sem)

        def wait_stage(j):
            pltpu.make_async_copy(ct_hbm.at[j], ring_slot(j),
                                  stage_sem).wait()

        def start_idx(j, b):
            pltpu.async_copy(ring_slot(j), idx_bufs[b], idx_sems[b])

        def wait_idx(j, b):
            pltpu.make_async_copy(ring_slot(j), idx_bufs[b],
                                  idx_sems[b]).wait()

        def start_out(j, b):
            pltpu.async_copy(out_bufs[b], out_hbm.at[j, k], out_sems[b])

        def wait_out(j, b):
            pltpu.make_async_copy(out_bufs[b], out_hbm.at[j, k],
                                  out_sems[b]).wait()

        def gather_row(b):
            idx_v, out_v = idx_bufs[b], out_bufs[b]

            @plsc.parallel_loop(0, n_groups, unroll=8)
            def _(g):
                idx16 = idx_v[pl.ds(g * L, L)]
                out_v[pl.ds(g * L, L)] = plsc.load_gather(row_v, [idx16])

        @pl.when(s == 0)
        def _():
            for j in range(2 * _LEAD - 1):
                start_stage(j)

        pltpu.sync_copy(table_hbm.at[k], row_v)

        @pl.when(s == 0)
        def _():
            for j in range(_NBUF - 1):
                wait_stage(j)

        plsc.subcore_barrier()
        for j in range(_NBUF - 1):
            start_idx(j, j)

        def body(i, carry):
            for p in range(2 * _NBUF):
                j = 2 * _NBUF * i + p
                b = p % _NBUF

                @pl.when((s == 0) & (j + _LEAD - 1 < B1))
                def _():
                    wait_stage(j + _LEAD - 1)

                plsc.subcore_barrier()

                @pl.when((s == 0) & (j + 2 * _LEAD - 1 < B1))
                def _():
                    start_stage(j + 2 * _LEAD - 1)

                @pl.when(j + _NBUF - 1 < B1)
                def _():
                    start_idx(j + _NBUF - 1, (p + _NBUF - 1) % _NBUF)

                wait_idx(j, b)

                if p >= _NBUF:
                    wait_out(j - _NBUF, b)
                else:
                    @pl.when(i > 0)
                    def _():
                        wait_out(j - _NBUF, b)

                gather_row(b)
                start_out(j, b)
            return carry

        n_iters = B1 // (2 * _NBUF)
        lax.fori_loop(0, n_iters, body, 0)
        rem = B1 - n_iters * 2 * _NBUF
        for p in range(rem):
            j = n_iters * 2 * _NBUF + p
            b = j % _NBUF

            @pl.when((s == 0) & (j + _LEAD - 1 < B1))
            def _():
                wait_stage(j + _LEAD - 1)

            plsc.subcore_barrier()

            @pl.when(j + _NBUF - 1 < B1)
            def _():
                start_idx(j + _NBUF - 1, (j + _NBUF - 1) % _NBUF)

            wait_idx(j, b)
            wait_out(j - _NBUF, b)
            gather_row(b)
            start_out(j, b)
        for j in range(B1 - _NBUF, B1):
            wait_out(j, j % _NBUF)

    return gather_kernel


def kernel(cell_types, table):
    B0, B1 = cell_types.shape
    V, D = table.shape
    ct_t = jnp.transpose(cell_types.astype(jnp.int32))
    table_t = jnp.transpose(table)
    out_t = _make_gather_t(V, D, B0, B1)(table_t, ct_t)
    return jnp.transpose(out_t, (2, 0, 1))


# speedup vs baseline: 27.3265x; 1.0026x over previous
import functools

import jax
import jax.numpy as jnp
from jax import lax
from jax.experimental import pallas as pl
from jax.experimental.pallas import tpu as pltpu
from jax.experimental.pallas import tpu_sc as plsc

_NBUF = 3
_LEAD = 3
_RING = 6


def _make_gather_t(V, D, B0, B1):
    info = plsc.get_sparse_core_info()
    NC, NS, L = info.num_cores, info.num_subcores, info.num_lanes
    assert NC * NS == D and B0 % L == 0
    n_groups = B0 // L

    mesh = plsc.VectorSubcoreMesh(core_axis_name="c", subcore_axis_name="s")

    @functools.partial(
        pl.kernel,
        mesh=mesh,
        out_type=jax.ShapeDtypeStruct((B1, D, B0), jnp.float32),
        scratch_types=[
            pltpu.VMEM((V,), jnp.float32),
            [pltpu.VMEM((B0,), jnp.int32) for _ in range(_NBUF)],
            [pltpu.VMEM((B0,), jnp.float32) for _ in range(_NBUF)],
            pltpu.VMEM_SHARED((_RING * B0,), jnp.int32),
            [pltpu.SemaphoreType.DMA for _ in range(_NBUF)],
            [pltpu.SemaphoreType.DMA for _ in range(_NBUF)],
            pltpu.SemaphoreType.DMA,
        ],
        compiler_params=pltpu.CompilerParams(use_tc_tiling_on_sc=True,
                                             needs_layout_passes=False),
    )
    def gather_kernel(table_hbm, ct_hbm, out_hbm, row_v, idx_bufs, out_bufs,
                      ct_ring, idx_sems, out_sems, stage_sem):
        s = lax.axis_index("s")
        k = lax.axis_index("c") * NS + s

        def ring_slot(j):
            return ct_ring.at[pl.ds((j % _RING) * B0, B0)]

        def start_stage(j):
            pltpu.async_copy(ct_hbm.at[j], ring_slot(j), stage_sem)

        def wait_stage(j):
            pltpu.make_async_copy(ct_hbm.at[j], ring_slot(j),
                                  stage_sem).wait()

        def start_idx(j, b):
            pltpu.async_copy(ring_slot(j), idx_bufs[b], idx_sems[b])

        def wait_idx(j, b):
            pltpu.make_async_copy(ring_slot(j), idx_bufs[b],
                                  idx_sems[b]).wait()

        def start_out(j, b):
            pltpu.async_copy(out_bufs[b], out_hbm.at[j, k], out_sems[b])

        def wait_out(j, b):
            pltpu.make_async_copy(out_bufs[b], out_hbm.at[j, k],
                                  out_sems[b]).wait()

        def gather_row(b):
            idx_v, out_v = idx_bufs[b], out_bufs[b]

            @plsc.parallel_loop(0, n_groups, unroll=16)
            def _(g):
                idx16 = idx_v[pl.ds(g * L, L)]
                out_v[pl.ds(g * L, L)] = plsc.load_gather(row_v, [idx16])

        @pl.when(s == 0)
        def _():
            for j in range(2 * _LEAD - 1):
                start_stage(j)

        pltpu.sync_copy(table_hbm.at[k], row_v)

        @pl.when(s == 0)
        def _():
            for j in range(_NBUF - 1):
                wait_stage(j)

        plsc.subcore_barrier()
        for j in range(_NBUF - 1):
            start_idx(j, j)

        def body(i, carry):
            for p in range(2 * _NBUF):
                j = 2 * _NBUF * i + p
                b = p % _NBUF

                @pl.when((s == 0) & (j + _LEAD - 1 < B1))
                def _():
                    wait_stage(j + _LEAD - 1)

                plsc.subcore_barrier()

                @pl.when((s == 0) & (j + 2 * _LEAD - 1 < B1))
                def _():
                    start_stage(j + 2 * _LEAD - 1)

                @pl.when(j + _NBUF - 1 < B1)
                def _():
                    start_idx(j + _NBUF - 1, (p + _NBUF - 1) % _NBUF)

                wait_idx(j, b)

                if p >= _NBUF:
                    wait_out(j - _NBUF, b)
                else:
                    @pl.when(i > 0)
                    def _():
                        wait_out(j - _NBUF, b)

                gather_row(b)
                start_out(j, b)
            return carry

        n_iters = B1 // (2 * _NBUF)
        lax.fori_loop(0, n_iters, body, 0)
        rem = B1 - n_iters * 2 * _NBUF
        for p in range(rem):
            j = n_iters * 2 * _NBUF + p
            b = j % _NBUF

            @pl.when((s == 0) & (j + _LEAD - 1 < B1))
            def _():
                wait_stage(j + _LEAD - 1)

            plsc.subcore_barrier()

            @pl.when(j + _NBUF - 1 < B1)
            def _():
                start_idx(j + _NBUF - 1, (j + _NBUF - 1) % _NBUF)

            wait_idx(j, b)
            wait_out(j - _NBUF, b)
            gather_row(b)
            start_out(j, b)
        for j in range(B1 - _NBUF, B1):
            wait_out(j, j % _NBUF)

    return gather_kernel


def kernel(cell_types, table):
    B0, B1 = cell_types.shape
    V, D = table.shape
    ct_t = jnp.transpose(cell_types.astype(jnp.int32))
    table_t = jnp.transpose(table)
    out_t = _make_gather_t(V, D, B0, B1)(table_t, ct_t)
    return jnp.transpose(out_t, (2, 0, 1))
